# baseline (speedup 1.0000x reference)
import functools

import jax
import jax.numpy as jnp
from jax import lax
from jax.experimental import pallas as pl
from jax.experimental.pallas import tpu as pltpu
from jax.experimental.pallas import tpu_sc as plsc

_NC, _NS = 2, 16
_NW = _NC * _NS

_VOCAB = 1000000
_DIM = 64
_B_FLAT = 4096 * 200
_CHUNK = 800


@functools.partial(jax.jit, static_argnums=())
def _gather(idx_flat, table):
    b_per_w = _B_FLAT // _NW
    n_chunks = b_per_w // _CHUNK
    mesh = plsc.VectorSubcoreMesh(core_axis_name="c", subcore_axis_name="s")

    @functools.partial(
        pl.kernel,
        out_type=jax.ShapeDtypeStruct((_B_FLAT, _DIM), jnp.float32),
        mesh=mesh,
        scratch_types=[
            pltpu.VMEM((b_per_w,), jnp.int32),
            pltpu.VMEM((_CHUNK, _DIM), jnp.float32),
            pltpu.SemaphoreType.DMA,
        ],
        compiler_params=pltpu.CompilerParams(use_tc_tiling_on_sc=False),
    )
    def gather_kernel(idx_hbm, table_hbm, out_hbm, idx_v, rows_v, sem):
        wid = lax.axis_index("s") * _NC + lax.axis_index("c")
        base = wid * b_per_w
        pltpu.sync_copy(idx_hbm.at[pl.ds(base, b_per_w)], idx_v)

        def body(c, carry):
            pltpu.async_copy(
                table_hbm.at[idx_v.at[pl.ds(c * _CHUNK, _CHUNK)]], rows_v, sem
            ).wait()
            pltpu.sync_copy(rows_v, out_hbm.at[pl.ds(base + c * _CHUNK, _CHUNK)])
            return carry

        lax.fori_loop(0, n_chunks, body, 0)

    return gather_kernel(idx_flat, table)


def kernel(batchinput, table):
    idx_flat = batchinput.reshape(-1).astype(jnp.int32)
    out = _gather(idx_flat, table)
    return out.reshape(batchinput.shape[0], batchinput.shape[1], _DIM)


# speedup vs baseline: 1.0086x; 1.0086x over previous
import functools

import jax
import jax.numpy as jnp
from jax import lax
from jax.experimental import pallas as pl
from jax.experimental.pallas import tpu as pltpu
from jax.experimental.pallas import tpu_sc as plsc

_NC, _NS = 2, 16
_NW = _NC * _NS

_VOCAB = 1000000
_DIM = 64
_B_FLAT = 4096 * 200
_CHUNK = 800


@functools.partial(jax.jit, static_argnums=())
def _gather(idx_flat, table):
    b_per_w = _B_FLAT // _NW
    n_chunks = b_per_w // _CHUNK
    n_pairs = n_chunks // 2
    mesh = plsc.VectorSubcoreMesh(core_axis_name="c", subcore_axis_name="s")

    @functools.partial(
        pl.kernel,
        out_type=jax.ShapeDtypeStruct((_B_FLAT, _DIM), jnp.float32),
        mesh=mesh,
        scratch_types=[
            pltpu.VMEM((b_per_w,), jnp.int32),
            pltpu.VMEM((_CHUNK, _DIM), jnp.float32),
            pltpu.VMEM((_CHUNK, _DIM), jnp.float32),
            pltpu.SemaphoreType.DMA,
            pltpu.SemaphoreType.DMA,
            pltpu.SemaphoreType.DMA,
            pltpu.SemaphoreType.DMA,
        ],
        compiler_params=pltpu.CompilerParams(use_tc_tiling_on_sc=False),
    )
    def gather_kernel(idx_hbm, table_hbm, out_hbm, idx_v, buf0, buf1, g0, g1, o0, o1):
        wid = lax.axis_index("s") * _NC + lax.axis_index("c")
        base = wid * b_per_w
        pltpu.sync_copy(idx_hbm.at[pl.ds(base, b_per_w)], idx_v)

        def gather_start(c, buf, sem):
            return pltpu.async_copy(
                table_hbm.at[idx_v.at[pl.ds(c * _CHUNK, _CHUNK)]], buf, sem
            )

        gather_start(0, buf0, g0)
        gather_start(1, buf1, g1)

        def gather_wait(buf, sem):
            pltpu.make_async_copy(
                table_hbm.at[idx_v.at[pl.ds(0, _CHUNK)]], buf, sem
            ).wait()

        def pair_body(i, carry):
            c = 2 * i
            gather_wait(buf0, g0)
            out0 = pltpu.async_copy(buf0, out_hbm.at[pl.ds(base + c * _CHUNK, _CHUNK)], o0)
            gather_wait(buf1, g1)
            out1 = pltpu.async_copy(buf1, out_hbm.at[pl.ds(base + (c + 1) * _CHUNK, _CHUNK)], o1)

            @pl.when(i < n_pairs - 1)
            def _():
                out0.wait()
                gather_start(c + 2, buf0, g0)
                out1.wait()
                gather_start(c + 3, buf1, g1)

            @pl.when(i == n_pairs - 1)
            def _():
                out0.wait()
                out1.wait()

            return carry

        lax.fori_loop(0, n_pairs, pair_body, 0)

    return gather_kernel(idx_flat, table)


def kernel(batchinput, table):
    idx_flat = batchinput.reshape(-1).astype(jnp.int32)
    out = _gather(idx_flat, table)
    return out.reshape(batchinput.shape[0], batchinput.shape[1], _DIM)


# speedup vs baseline: 1.0166x; 1.0080x over previous
import functools

import jax
import jax.numpy as jnp
from jax import lax
from jax.experimental import pallas as pl
from jax.experimental.pallas import tpu as pltpu
from jax.experimental.pallas import tpu_sc as plsc

_NC, _NS = 2, 16
_NW = _NC * _NS

_VOCAB = 1000000
_DIM = 64
_BATCH = 4096
_HIST = 200
_B_FLAT = _BATCH * _HIST

_CB = 2
_NBUF = 4


@jax.jit
def _gather(idx_flat, table):
    b_per_w = _B_FLAT // _NW
    batches_per_w = _BATCH // _NW
    chunk = _CB * _HIST
    n_chunks = batches_per_w // _CB
    n_rounds = n_chunks // _NBUF
    mesh = plsc.VectorSubcoreMesh(core_axis_name="c", subcore_axis_name="s")

    scratch = [pltpu.VMEM((b_per_w,), jnp.int32)]
    scratch += [pltpu.VMEM((chunk, _DIM), jnp.float32) for _ in range(_NBUF)]
    scratch += [pltpu.SemaphoreType.DMA for _ in range(2 * _NBUF)]

    @functools.partial(
        pl.kernel,
        out_type=jax.ShapeDtypeStruct((_BATCH, _HIST, _DIM), jnp.float32),
        mesh=mesh,
        scratch_types=scratch,
        compiler_params=pltpu.CompilerParams(use_tc_tiling_on_sc=False),
    )
    def gather_kernel(idx_hbm, table_hbm, out_hbm, idx_v, *bufs_and_sems):
        bufs = bufs_and_sems[:_NBUF]
        gsems = bufs_and_sems[_NBUF : 2 * _NBUF]
        osems = bufs_and_sems[2 * _NBUF :]

        wid = lax.axis_index("s") * _NC + lax.axis_index("c")
        base = wid * b_per_w
        bbase = wid * batches_per_w
        pltpu.sync_copy(idx_hbm.at[pl.ds(base, b_per_w)], idx_v)

        def gather_start(c, k):
            return pltpu.async_copy(
                table_hbm.at[idx_v.at[pl.ds(c * chunk, chunk)]], bufs[k], gsems[k]
            )

        def gather_wait(k):
            pltpu.make_async_copy(
                table_hbm.at[idx_v.at[pl.ds(0, chunk)]], bufs[k], gsems[k]
            ).wait()

        for k in range(_NBUF):
            gather_start(k, k)

        def round_body(r, carry):
            for k in range(_NBUF):
                c = r * _NBUF + k
                gather_wait(k)
                outs = []
                for j in range(_CB):
                    outs.append(
                        pltpu.async_copy(
                            bufs[k].at[pl.ds(j * _HIST, _HIST)],
                            out_hbm.at[bbase + c * _CB + j],
                            osems[k],
                        )
                    )
                for d in outs:
                    d.wait()

                @pl.when(c + _NBUF < n_chunks)
                def _():
                    gather_start(c + _NBUF, k)

            return carry

        lax.fori_loop(0, n_rounds, round_body, 0)

    return gather_kernel(idx_flat, table)


def kernel(batchinput, table):
    idx_flat = batchinput.reshape(-1).astype(jnp.int32)
    return _gather(idx_flat, table)


# speedup vs baseline: 1.2410x; 1.2207x over previous
import functools

import jax
import jax.numpy as jnp
from jax import lax
from jax.experimental import pallas as pl
from jax.experimental.pallas import tpu as pltpu
from jax.experimental.pallas import tpu_sc as plsc

_NC, _NS = 2, 16
_NW = _NC * _NS

_VOCAB = 1000000
_DIM = 64
_PDIM = 128
_BATCH = 4096
_HIST = 200
_B_FLAT = _BATCH * _HIST

_NBUF = 4


@jax.jit
def _gather(idx_flat, table128):
    b_per_w = _B_FLAT // _NW
    batches_per_w = _BATCH // _NW
    n_chunks = batches_per_w
    n_rounds = n_chunks // _NBUF
    mesh = plsc.VectorSubcoreMesh(core_axis_name="c", subcore_axis_name="s")

    scratch = [pltpu.VMEM((b_per_w,), jnp.int32)]
    scratch += [pltpu.VMEM((_HIST, _PDIM), jnp.float32) for _ in range(_NBUF)]
    scratch += [pltpu.SemaphoreType.DMA for _ in range(2 * _NBUF)]

    @functools.partial(
        pl.kernel,
        out_type=jax.ShapeDtypeStruct((_BATCH, _HIST, _PDIM), jnp.float32),
        mesh=mesh,
        scratch_types=scratch,
    )
    def gather_kernel(idx_hbm, table_hbm, out_hbm, idx_v, *bufs_and_sems):
        bufs = bufs_and_sems[:_NBUF]
        gsems = bufs_and_sems[_NBUF : 2 * _NBUF]
        osems = bufs_and_sems[2 * _NBUF :]

        wid = lax.axis_index("s") * _NC + lax.axis_index("c")
        base = wid * b_per_w
        bbase = wid * batches_per_w
        pltpu.sync_copy(idx_hbm.at[pl.ds(base, b_per_w)], idx_v)

        def gather_start(c, k):
            return pltpu.async_copy(
                table_hbm.at[idx_v.at[pl.ds(c * _HIST, _HIST)]], bufs[k], gsems[k]
            )

        def gather_wait(k):
            pltpu.make_async_copy(
                table_hbm.at[idx_v.at[pl.ds(0, _HIST)]], bufs[k], gsems[k]
            ).wait()

        for k in range(_NBUF):
            gather_start(k, k)

        def round_body(r, carry):
            for k in range(_NBUF):
                c = r * _NBUF + k
                gather_wait(k)
                pltpu.async_copy(bufs[k], out_hbm.at[bbase + c], osems[k]).wait()

                @pl.when(c + _NBUF < n_chunks)
                def _():
                    gather_start(c + _NBUF, k)

            return carry

        lax.fori_loop(0, n_rounds, round_body, 0)

    return gather_kernel(idx_flat, table128)


def kernel(batchinput, table):
    idx_flat = batchinput.reshape(-1).astype(jnp.int32)
    table128 = jnp.pad(table, ((0, 0), (0, _PDIM - _DIM)))
    out128 = _gather(idx_flat, table128)
    return out128[:, :, :_DIM]


# speedup vs baseline: 1.4460x; 1.1652x over previous
import functools

import jax
import jax.numpy as jnp
from jax import lax
from jax.experimental import pallas as pl
from jax.experimental.pallas import tpu as pltpu
from jax.experimental.pallas import tpu_sc as plsc

_NC, _NS = 2, 16
_NW = _NC * _NS

_VOCAB = 1000000
_DIM = 64
_PDIM = 128
_BATCH = 4096
_HIST = 200
_B_FLAT = _BATCH * _HIST

_NBUF = 4


@jax.jit
def _gather(idx2_flat, table2):
    b_per_w = _B_FLAT // _NW
    batches_per_w = _BATCH // _NW
    n_chunks = batches_per_w
    n_rounds = n_chunks // _NBUF
    mesh = plsc.VectorSubcoreMesh(core_axis_name="c", subcore_axis_name="s")

    scratch = [pltpu.VMEM((b_per_w,), jnp.int32)]
    scratch += [pltpu.VMEM((_HIST, _DIM), jnp.float32) for _ in range(_NBUF)]
    scratch += [pltpu.SemaphoreType.DMA for _ in range(2 * _NBUF)]

    @functools.partial(
        pl.kernel,
        out_type=jax.ShapeDtypeStruct((_BATCH, _HIST, _PDIM), jnp.float32),
        mesh=mesh,
        scratch_types=scratch,
        compiler_params=pltpu.CompilerParams(use_tc_tiling_on_sc=False),
    )
    def gather_kernel(idx_hbm, table_hbm, out_hbm, idx_v, *bufs_and_sems):
        bufs = bufs_and_sems[:_NBUF]
        gsems = bufs_and_sems[_NBUF : 2 * _NBUF]
        osems = bufs_and_sems[2 * _NBUF :]

        wid = lax.axis_index("s") * _NC + lax.axis_index("c")
        base = wid * b_per_w
        bbase = wid * batches_per_w
        pltpu.sync_copy(idx_hbm.at[pl.ds(base, b_per_w)], idx_v)

        def gather_start(c, k):
            return pltpu.async_copy(
                table_hbm.at[idx_v.at[pl.ds(c * _HIST, _HIST)]], bufs[k], gsems[k]
            )

        def gather_wait(k):
            pltpu.make_async_copy(
                table_hbm.at[idx_v.at[pl.ds(0, _HIST)]], bufs[k], gsems[k]
            ).wait()

        for k in range(_NBUF):
            gather_start(k, k)

        def round_body(r, carry):
            for k in range(_NBUF):
                c = r * _NBUF + k
                gather_wait(k)
                pltpu.async_copy(
                    bufs[k],
                    out_hbm.at[bbase + c].at[:, pl.ds(0, _DIM)],
                    osems[k],
                ).wait()

                @pl.when(c + _NBUF < n_chunks)
                def _():
                    gather_start(c + _NBUF, k)

            return carry

        lax.fori_loop(0, n_rounds, round_body, 0)

    return gather_kernel(idx2_flat, table2)


def kernel(batchinput, table):
    idx2_flat = batchinput.reshape(-1).astype(jnp.int32) * 2
    table2 = jnp.pad(table, ((0, 0), (0, _PDIM - _DIM))).reshape(2 * _VOCAB, _DIM)
    out128 = _gather(idx2_flat, table2)
    return out128[:, :, :_DIM]
